# Initial kernel scaffold; baseline (speedup 1.0000x reference)
import jax
import jax.numpy as jnp
from jax.experimental import pallas as pl


def kernel(x, edge_index, W1, b1, W2, b2):
    raise NotImplementedError("write your pallas kernel here")


# speedup vs baseline: 11.0434x; 11.0434x over previous
import functools

import jax
import jax.numpy as jnp
from jax import lax
from jax.experimental import pallas as pl
from jax.experimental.pallas import tpu as pltpu
from jax.experimental.pallas import tpu_sc as plsc

N = 10000
E = 320000
F_IN = 128
HID = 64
CLS = 47
K = 10
ALPHA = 0.1

NPAD = 10240
CPAD = 48
NTILES = 32
BATCH = 128
EPT = 10240
NB = EPT // BATCH
ROWS_PER_SUB = NPAD // 16

_MESH = plsc.VectorSubcoreMesh(core_axis_name="c", subcore_axis_name="s")
_SC_PARAMS = pltpu.CompilerParams(use_tc_tiling_on_sc=False)


def _fill_f32(ref, value, total):
    vec = jnp.full((16,), value, dtype=jnp.float32)

    def body(i, _):
        ref[pl.ds(i * 16, 16)] = vec
        return 0

    lax.fori_loop(0, total // 16, body, 0)


@functools.partial(
    pl.kernel,
    out_type=jax.ShapeDtypeStruct((2, NPAD), jnp.float32),
    mesh=_MESH,
    scratch_types=[
        pltpu.VMEM((NB, BATCH), jnp.int32),
        pltpu.VMEM((BATCH,), jnp.float32),
        pltpu.VMEM((ROWS_PER_SUB,), jnp.float32),
        pltpu.VMEM_SHARED((NPAD,), jnp.float32),
        pltpu.SemaphoreType.DMA,
    ],
    compiler_params=_SC_PARAMS,
)
def _deg_kernel(dst_hbm, out_hbm, idx_v, ones_v, zeros_v, acc_sh, sem):
    cid = lax.axis_index("c")
    sid = lax.axis_index("s")
    wid = cid * 16 + sid
    _fill_f32(zeros_v, 0.0, ROWS_PER_SUB)
    _fill_f32(ones_v, 1.0, BATCH)
    pltpu.sync_copy(zeros_v, acc_sh.at[pl.ds(sid * ROWS_PER_SUB, ROWS_PER_SUB)])
    plsc.subcore_barrier()
    pltpu.async_copy(dst_hbm.at[wid], idx_v, sem).wait()

    def body(j, _):
        pltpu.sync_copy(ones_v, acc_sh.at[idx_v.at[j]], add=True)
        return 0

    lax.fori_loop(0, NB, body, 0)
    plsc.subcore_barrier()
    sl = pl.ds(sid * ROWS_PER_SUB, ROWS_PER_SUB)
    pltpu.sync_copy(acc_sh.at[sl], out_hbm.at[cid, sl])


@functools.partial(
    pl.kernel,
    out_type=jax.ShapeDtypeStruct((2, NPAD, CPAD), jnp.float32),
    mesh=_MESH,
    scratch_types=[
        pltpu.VMEM((NB, BATCH), jnp.int32),
        pltpu.VMEM((NB, BATCH), jnp.int32),
        pltpu.VMEM((BATCH, CPAD), jnp.float32),
        pltpu.VMEM((BATCH, CPAD), jnp.float32),
        pltpu.VMEM((BATCH, CPAD), jnp.float32),
        pltpu.VMEM_SHARED((NPAD, CPAD), jnp.float32),
        pltpu.SemaphoreType.DMA,
        pltpu.SemaphoreType.DMA,
        pltpu.SemaphoreType.DMA,
    ],
    compiler_params=_SC_PARAMS,
)
def _prop_kernel(cur_hbm, src_hbm, dst_hbm, out_hbm,
                 src_v, dst_v, rows0, rows1, zeros_v, acc_sh,
                 sem0, sem1, semi):
    cid = lax.axis_index("c")
    sid = lax.axis_index("s")
    wid = cid * 16 + sid

    def zfill(r, _):
        zeros_v[r, pl.ds(0, 16)] = jnp.zeros((16,), jnp.float32)
        zeros_v[r, pl.ds(16, 16)] = jnp.zeros((16,), jnp.float32)
        zeros_v[r, pl.ds(32, 16)] = jnp.zeros((16,), jnp.float32)
        return 0

    lax.fori_loop(0, BATCH, zfill, 0)
    base = sid * ROWS_PER_SUB
    for t in range(ROWS_PER_SUB // BATCH):
        pltpu.sync_copy(zeros_v, acc_sh.at[pl.ds(base + t * BATCH, BATCH)])

    pltpu.async_copy(src_hbm.at[wid], src_v, semi).wait()
    pltpu.async_copy(dst_hbm.at[wid], dst_v, semi).wait()
    plsc.subcore_barrier()

    rows = (rows0, rows1)
    sems = (sem0, sem1)
    pltpu.async_copy(cur_hbm.at[src_v.at[0]], rows0, sem0)
    pltpu.async_copy(cur_hbm.at[src_v.at[1]], rows1, sem1)

    def body(jj, _):
        for b in range(2):
            j = jj * 2 + b
            pltpu.make_async_copy(cur_hbm.at[src_v.at[j]], rows[b], sems[b]).wait()
            pltpu.sync_copy(rows[b], acc_sh.at[dst_v.at[j]], add=True)
            @pl.when(j + 2 < NB)
            def _():
                pltpu.async_copy(cur_hbm.at[src_v.at[j + 2]], rows[b], sems[b])
        return 0

    lax.fori_loop(0, NB // 2, body, 0)
    plsc.subcore_barrier()
    sl = pl.ds(sid * ROWS_PER_SUB, ROWS_PER_SUB)
    pltpu.sync_copy(acc_sh.at[sl], out_hbm.at[cid, sl])


def _mlp_body(x_ref, w1_ref, b1_ref, w2_ref, b2_ref, deg_ref,
              u0_ref, dinvsq_ref):
    h = jnp.maximum(
        jnp.dot(x_ref[...], w1_ref[...], preferred_element_type=jnp.float32)
        + b1_ref[...][None, :], 0.0)
    h = jnp.dot(h, w2_ref[...], preferred_element_type=jnp.float32) \
        + b2_ref[...][None, :]
    deg = deg_ref[0, :] + deg_ref[1, :] + 1.0
    dinv = lax.rsqrt(deg)
    u0_ref[...] = h * dinv[:, None]
    dinvsq_ref[...] = 1.0 / deg


def _mlp(x_pad, W1, b1, W2p, b2p, deg_part):
    blk = 512
    grid = NPAD // blk
    return pl.pallas_call(
        _mlp_body,
        grid=(grid,),
        in_specs=[
            pl.BlockSpec((blk, F_IN), lambda i: (i, 0)),
            pl.BlockSpec((F_IN, HID), lambda i: (0, 0)),
            pl.BlockSpec((HID,), lambda i: (0,)),
            pl.BlockSpec((HID, CPAD), lambda i: (0, 0)),
            pl.BlockSpec((CPAD,), lambda i: (0,)),
            pl.BlockSpec((2, blk), lambda i: (0, i)),
        ],
        out_specs=[
            pl.BlockSpec((blk, CPAD), lambda i: (i, 0)),
            pl.BlockSpec((blk,), lambda i: (i,)),
        ],
        out_shape=[
            jax.ShapeDtypeStruct((NPAD, CPAD), jnp.float32),
            jax.ShapeDtypeStruct((NPAD,), jnp.float32),
        ],
    )(x_pad, W1, b1, W2p, b2p, deg_part)


def _combine_body(part_ref, u_ref, dinvsq_ref, out_ref):
    s = part_ref[0] + part_ref[1] + u_ref[...]
    out_ref[...] = s * dinvsq_ref[...][:, None]


def _combine(part, u, dinvsq):
    blk = 512
    grid = NPAD // blk
    return pl.pallas_call(
        _combine_body,
        grid=(grid,),
        in_specs=[
            pl.BlockSpec((2, blk, CPAD), lambda i: (0, i, 0)),
            pl.BlockSpec((blk, CPAD), lambda i: (i, 0)),
            pl.BlockSpec((blk,), lambda i: (i,)),
        ],
        out_specs=pl.BlockSpec((blk, CPAD), lambda i: (i, 0)),
        out_shape=jax.ShapeDtypeStruct((NPAD, CPAD), jnp.float32),
    )(part, u, dinvsq)


def _final_body(coefs, *refs):
    us = refs[:-2]
    dinvsq_ref = refs[-2]
    out_ref = refs[-1]
    acc = coefs[0] * us[0][...]
    for c, u in zip(coefs[1:], us[1:]):
        acc = acc + c * u[...]
    v = acc * lax.rsqrt(dinvsq_ref[...])
    col = lax.broadcasted_iota(jnp.int32, v.shape, 1)
    valid = col < CLS
    neg = jnp.full_like(v, -jnp.inf)
    m = jnp.max(jnp.where(valid, v, neg), axis=1, keepdims=True)
    ex = jnp.where(valid, jnp.exp(v - m), 0.0)
    s = jnp.sum(ex, axis=1, keepdims=True)
    res = v - m - jnp.log(s)
    out_ref[...] = res[:, :CLS]


def _final(us, dinvsq, coefs):
    blk = 400
    grid = N // blk
    return pl.pallas_call(
        functools.partial(_final_body, coefs),
        grid=(grid,),
        in_specs=[pl.BlockSpec((blk, CPAD), lambda i: (i, 0)) for _ in us]
        + [pl.BlockSpec((blk, 1), lambda i: (i, 0))],
        out_specs=pl.BlockSpec((blk, CLS), lambda i: (i, 0)),
        out_shape=jax.ShapeDtypeStruct((N, CLS), jnp.float32),
    )(*us, dinvsq[:, None])


def kernel(x, edge_index, W1, b1, W2, b2):
    src = edge_index[0].astype(jnp.int32)
    dst = edge_index[1].astype(jnp.int32)
    epad = NTILES * EPT - E
    src = jnp.concatenate([src, jnp.zeros((epad,), jnp.int32)])
    dst = jnp.concatenate([dst, jnp.full((epad,), NPAD - 1, jnp.int32)])
    src_t = src.reshape(NTILES, NB, BATCH)
    dst_t = dst.reshape(NTILES, NB, BATCH)

    x_pad = jnp.pad(x, ((0, NPAD - N), (0, 0)))
    W2p = jnp.pad(W2, ((0, 0), (0, CPAD - CLS)))
    b2p = jnp.pad(b2, ((0, CPAD - CLS),))

    deg_part = _deg_kernel(dst_t)
    u, dinvsq = _mlp(x_pad, W1, b1, W2p, b2p, deg_part)

    khalf = K // 2
    coef = [ALPHA * (1.0 - ALPHA) ** i for i in range(khalf + 1)]
    coef[khalf] = (1.0 - ALPHA) ** khalf

    evens = [u]
    for _ in range(khalf):
        for _ in range(2):
            part = _prop_kernel(u, src_t, dst_t)
            u = _combine(part, u, dinvsq)
        evens.append(u)

    return _final(evens, dinvsq, coef)
